# Initial kernel scaffold; baseline (speedup 1.0000x reference)
import jax
import jax.numpy as jnp
from jax.experimental import pallas as pl


def kernel(x, edge_index, W1, b1, W2, b2, W3, b3, W4, b4, W_emb):
    raise NotImplementedError("write your pallas kernel here")


# speedup vs baseline: 9.1338x; 9.1338x over previous
import functools

import jax
import jax.numpy as jnp
from jax import lax
from jax.experimental import pallas as pl
from jax.experimental.pallas import tpu as pltpu
from jax.experimental.pallas import tpu_sc as plsc

N = 10000
E = 320000
D = 128
NC = 2
NS = 16
NW = NC * NS
KB = 128
BPT = 79
EPAD = NW * BPT * KB
NP = 10240
RPT = NP // NS


_MESH = plsc.VectorSubcoreMesh(core_axis_name="c", subcore_axis_name="s")


@functools.partial(
    pl.kernel,
    out_type=jax.ShapeDtypeStruct((NC, NP), jnp.float32),
    mesh=_MESH,
    scratch_types=[
        pltpu.VMEM((BPT, KB), jnp.int32),
        pltpu.VMEM((KB,), jnp.float32),
        pltpu.VMEM_SHARED((NP,), jnp.float32),
    ],
)
def _deg(dst_hbm, ones_hbm, z_hbm, out_hbm, dstbuf, ones_v, acc):
    c = lax.axis_index("c")
    s = lax.axis_index("s")
    wid = c * NS + s
    pltpu.sync_copy(z_hbm.at[pl.ds(0, RPT)], acc.at[pl.ds(s * RPT, RPT)])
    pltpu.sync_copy(ones_hbm, ones_v)
    pltpu.sync_copy(dst_hbm.at[wid], dstbuf)
    plsc.subcore_barrier()

    def body(b, carry):
        pltpu.sync_copy(ones_v, acc.at[dstbuf.at[b]], add=True)
        return carry

    lax.fori_loop(0, BPT, body, 0)
    plsc.subcore_barrier()
    pltpu.sync_copy(acc.at[pl.ds(s * RPT, RPT)],
                    out_hbm.at[c, pl.ds(s * RPT, RPT)])


@functools.partial(
    pl.kernel,
    out_type=jax.ShapeDtypeStruct((NC, NP, D), jnp.float32),
    mesh=_MESH,
    scratch_types=[
        pltpu.VMEM((BPT, KB), jnp.int32),
        pltpu.VMEM((BPT, KB), jnp.int32),
        pltpu.VMEM((KB, D), jnp.float32),
        pltpu.VMEM_SHARED((NP, D), jnp.float32),
        pltpu.SemaphoreType.DMA,
    ],
)
def _agg(g_hbm, src_hbm, dst_hbm, z_hbm, out_hbm,
         srcbuf, dstbuf, rows, acc, sem):
    c = lax.axis_index("c")
    s = lax.axis_index("s")
    wid = c * NS + s
    pltpu.sync_copy(z_hbm, acc.at[pl.ds(s * RPT, RPT)])
    pltpu.sync_copy(src_hbm.at[wid], srcbuf)
    pltpu.sync_copy(dst_hbm.at[wid], dstbuf)
    plsc.subcore_barrier()

    def body(b, carry):
        pltpu.async_copy(g_hbm.at[srcbuf.at[b]], rows, sem).wait()
        pltpu.sync_copy(rows, acc.at[dstbuf.at[b]], add=True)
        return carry

    lax.fori_loop(0, BPT, body, 0)
    plsc.subcore_barrier()
    pltpu.sync_copy(acc.at[pl.ds(s * RPT, RPT)],
                    out_hbm.at[c, pl.ds(s * RPT, RPT)])


def _norm_body(d_ref, o_ref):
    o_ref[...] = lax.rsqrt(d_ref[0] + d_ref[1] + 1.0)


_norm = pl.pallas_call(
    _norm_body,
    out_shape=jax.ShapeDtypeStruct((NP // D, D), jnp.float32),
)

_RB = 640
_GRID = NP // _RB


def _first_body(x_ref, c_ref, w_ref, o_ref):
    o_ref[...] = c_ref[...] * jnp.dot(
        x_ref[...], w_ref[...], preferred_element_type=jnp.float32)


_first = pl.pallas_call(
    _first_body,
    grid=(_GRID,),
    in_specs=[
        pl.BlockSpec((_RB, D), lambda i: (i, 0)),
        pl.BlockSpec((_RB, 1), lambda i: (i, 0)),
        pl.BlockSpec((D, D), lambda i: (0, 0)),
    ],
    out_specs=pl.BlockSpec((_RB, D), lambda i: (i, 0)),
    out_shape=jax.ShapeDtypeStruct((NP, D), jnp.float32),
)


def _layer_body(s_ref, g_ref, c_ref, b_ref, w_ref, o_ref):
    cb = c_ref[...]
    t = cb * (s_ref[0] + s_ref[1] + g_ref[...]) + b_ref[...]
    t = jnp.maximum(t, 0.0)
    o_ref[...] = cb * jnp.dot(t, w_ref[...],
                              preferred_element_type=jnp.float32)


_layer = pl.pallas_call(
    _layer_body,
    grid=(_GRID,),
    in_specs=[
        pl.BlockSpec((NC, _RB, D), lambda i: (0, i, 0)),
        pl.BlockSpec((_RB, D), lambda i: (i, 0)),
        pl.BlockSpec((_RB, 1), lambda i: (i, 0)),
        pl.BlockSpec((1, D), lambda i: (0, 0)),
        pl.BlockSpec((D, D), lambda i: (0, 0)),
    ],
    out_specs=pl.BlockSpec((_RB, D), lambda i: (i, 0)),
    out_shape=jax.ShapeDtypeStruct((NP, D), jnp.float32),
)


def _final_body(s_ref, g_ref, c_ref, b_ref, o_ref):
    o_ref[...] = (c_ref[...] * (s_ref[0] + s_ref[1] + g_ref[...])
                  + b_ref[...])


_final = pl.pallas_call(
    _final_body,
    grid=(_GRID,),
    in_specs=[
        pl.BlockSpec((NC, _RB, D), lambda i: (0, i, 0)),
        pl.BlockSpec((_RB, D), lambda i: (i, 0)),
        pl.BlockSpec((_RB, 1), lambda i: (i, 0)),
        pl.BlockSpec((1, D), lambda i: (0, 0)),
    ],
    out_specs=pl.BlockSpec((_RB, D), lambda i: (i, 0)),
    out_shape=jax.ShapeDtypeStruct((NP, D), jnp.float32),
)


def kernel(x, edge_index, W1, b1, W2, b2, W3, b3, W4, b4, W_emb):
    pad = EPAD - E
    src_r = jnp.concatenate(
        [edge_index[0], jnp.zeros((pad,), jnp.int32)]).reshape(NW, BPT, KB)
    dst_r = jnp.concatenate(
        [edge_index[1], jnp.full((pad,), N, jnp.int32)]).reshape(NW, BPT, KB)
    ones_k = jnp.ones((KB,), jnp.float32)
    zrows = jnp.zeros((RPT, D), jnp.float32)
    zvec = jnp.zeros((NP,), jnp.float32)

    deg2 = _deg(dst_r, ones_k, zvec)
    cmat = _norm(deg2.reshape(NC, NP // D, D))
    c = cmat.reshape(NP, 1)
    xp = jnp.concatenate([x, jnp.zeros((NP - N, D), jnp.float32)])

    g = _first(xp, c, W1)
    for (W, b) in ((W2, b1), (W3, b2), (W4, b3)):
        s2 = _agg(g, src_r, dst_r, zrows)
        g = _layer(s2, g, c, b.reshape(1, D), W)
    s2 = _agg(g, src_r, dst_r, zrows)
    out = _final(s2, g, c, b4.reshape(1, D))
    return out[:N].reshape(-1, 5, D)
